# baseline (speedup 1.0000x reference)
import functools

import jax
import jax.numpy as jnp
from jax.experimental import pallas as pl
from jax.experimental.pallas import tpu as pltpu

_N = 10000
_D = 128
_BM = 200


def _proj_body(feat_ref, w1_ref, g_ref):
    f = feat_ref[...].astype(jnp.bfloat16)
    w = w1_ref[...].astype(jnp.bfloat16)
    g = jax.lax.dot_general(
        f, w, (((1,), (0,)), ((), ())), preferred_element_type=jnp.float32
    )
    g_ref[...] = g.astype(jnp.bfloat16)


def _fused_body(adj_ref, g_ref, b1_ref, w2_ref, out_ref, h2_ref):
    p = pl.program_id(0)
    i = pl.program_id(1)
    a = adj_ref[...].astype(jnp.bfloat16)

    @pl.when(p == 0)
    def _phase0():
        acc = jax.lax.dot_general(
            a, g_ref[...], (((1,), (0,)), ((), ())),
            preferred_element_type=jnp.float32,
        )
        h = jnp.tanh(acc + b1_ref[...])
        h2 = jax.lax.dot_general(
            h.astype(jnp.bfloat16),
            w2_ref[...].astype(jnp.bfloat16),
            (((1,), (0,)), ((), ())),
            preferred_element_type=jnp.float32,
        )
        h2_ref[pl.ds(i * _BM, _BM), :] = h2.astype(jnp.bfloat16)

    @pl.when(p == 1)
    def _phase1():
        out_ref[...] = jax.lax.dot_general(
            a, h2_ref[...], (((1,), (0,)), ((), ())),
            preferred_element_type=jnp.float32,
        )


@jax.jit
def _run(adj, feat, W1, b1, W2):
    n, d, bm = _N, _D, _BM
    num_blocks = n // bm

    g = pl.pallas_call(
        _proj_body,
        grid=(10,),
        in_specs=[
            pl.BlockSpec((n // 10, d), lambda i: (i, 0)),
            pl.BlockSpec((d, d), lambda i: (0, 0)),
        ],
        out_specs=pl.BlockSpec((n // 10, d), lambda i: (i, 0)),
        out_shape=jax.ShapeDtypeStruct((n, d), jnp.bfloat16),
    )(feat, W1)

    b1_2d = b1.reshape(1, d)

    out = pl.pallas_call(
        _fused_body,
        grid=(2, num_blocks),
        in_specs=[
            pl.BlockSpec((bm, n), lambda p, i: (i, 0)),
            pl.BlockSpec((n, d), lambda p, i: (0, 0)),
            pl.BlockSpec((1, d), lambda p, i: (0, 0)),
            pl.BlockSpec((d, d), lambda p, i: (0, 0)),
        ],
        out_specs=pl.BlockSpec((bm, d), lambda p, i: (i, 0)),
        out_shape=jax.ShapeDtypeStruct((n, d), jnp.float32),
        scratch_shapes=[pltpu.VMEM((n, d), jnp.bfloat16)],
    )(adj, g, b1_2d, W2)
    return out


def kernel(adj, feat, W1, b1, W2):
    return _run(adj, feat, W1, b1, W2)


# speedup vs baseline: 1.1142x; 1.1142x over previous
import functools

import jax
import jax.numpy as jnp
from jax.experimental import pallas as pl
from jax.experimental.pallas import tpu as pltpu

_N = 10000
_D = 128
_BM = 400
_NB = _N // _BM
_K_STASH = 2


def _proj_body(feat_ref, w1_ref, g_ref):
    f = feat_ref[...].astype(jnp.bfloat16)
    w = w1_ref[...].astype(jnp.bfloat16)
    g = jax.lax.dot_general(
        f, w, (((1,), (0,)), ((), ())), preferred_element_type=jnp.float32
    )
    g_ref[...] = g.astype(jnp.bfloat16)


_CHUNKS = (0, 2560, 5120, 7680, 10000)


def _dot_bf16(a, b):
    return jax.lax.dot_general(
        a, b, (((1,), (0,)), ((), ())), preferred_element_type=jnp.float32
    )


def _fused_body(adj_ref, g_ref, b1_ref, w2_ref, out_ref, h2_ref, stash_ref):
    p = pl.program_id(0)
    i = pl.program_id(1)
    first_stashed = _NB - _K_STASH

    @pl.when(p == 0)
    def _phase0():
        acc = jnp.zeros((_BM, _D), jnp.float32)
        for c in range(len(_CHUNKS) - 1):
            lo, hi = _CHUNKS[c], _CHUNKS[c + 1]
            ab = adj_ref[:, lo:hi].astype(jnp.bfloat16)
            acc = acc + _dot_bf16(ab, g_ref[lo:hi, :])

            @pl.when(i >= first_stashed)
            def _save():
                stash_ref[pl.ds((i - first_stashed) * _BM, _BM), lo:hi] = ab

        h = jnp.tanh(acc + b1_ref[...])
        h2 = _dot_bf16(h.astype(jnp.bfloat16), w2_ref[...].astype(jnp.bfloat16))
        h2_ref[pl.ds(i * _BM, _BM), :] = h2.astype(jnp.bfloat16)

    @pl.when(jnp.logical_and(p == 1, i < first_stashed))
    def _phase1_stream():
        acc = jnp.zeros((_BM, _D), jnp.float32)
        for c in range(len(_CHUNKS) - 1):
            lo, hi = _CHUNKS[c], _CHUNKS[c + 1]
            ab = adj_ref[:, lo:hi].astype(jnp.bfloat16)
            acc = acc + _dot_bf16(ab, h2_ref[lo:hi, :])
        out_ref[...] = acc

    @pl.when(jnp.logical_and(p == 1, i >= first_stashed))
    def _phase1_stash():
        a = stash_ref[pl.ds((i - first_stashed) * _BM, _BM), :]
        out_ref[...] = _dot_bf16(a, h2_ref[...])


def _adj_index(p, i):
    first_stashed = _NB - _K_STASH
    streamed = jnp.where(
        jnp.logical_and(p == 1, i >= first_stashed), first_stashed - 1, i
    )
    return (streamed, 0)


@jax.jit
def _run(adj, feat, W1, b1, W2):
    n, d, bm = _N, _D, _BM

    g = pl.pallas_call(
        _proj_body,
        grid=(10,),
        in_specs=[
            pl.BlockSpec((n // 10, d), lambda i: (i, 0)),
            pl.BlockSpec((d, d), lambda i: (0, 0)),
        ],
        out_specs=pl.BlockSpec((n // 10, d), lambda i: (i, 0)),
        out_shape=jax.ShapeDtypeStruct((n, d), jnp.bfloat16),
    )(feat, W1)

    b1_2d = b1.reshape(1, d)

    out = pl.pallas_call(
        _fused_body,
        grid=(2, _NB),
        in_specs=[
            pl.BlockSpec((bm, n), _adj_index),
            pl.BlockSpec((n, d), lambda p, i: (0, 0)),
            pl.BlockSpec((1, d), lambda p, i: (0, 0)),
            pl.BlockSpec((d, d), lambda p, i: (0, 0)),
        ],
        out_specs=pl.BlockSpec((bm, d), lambda p, i: (i, 0)),
        out_shape=jax.ShapeDtypeStruct((n, d), jnp.float32),
        scratch_shapes=[
            pltpu.VMEM((n, d), jnp.bfloat16),
            pltpu.VMEM((_K_STASH * bm, n), jnp.bfloat16),
        ],
        compiler_params=pltpu.CompilerParams(
            vmem_limit_bytes=63 * 1024 * 1024,
        ),
    )(adj, g, b1_2d, W2)
    return out


def kernel(adj, feat, W1, b1, W2):
    return _run(adj, feat, W1, b1, W2)


# speedup vs baseline: 1.1193x; 1.0046x over previous
import functools

import jax
import jax.numpy as jnp
from jax.experimental import pallas as pl
from jax.experimental.pallas import tpu as pltpu

_N = 10000
_D = 128
_BM = 400
_NB = _N // _BM
_K_STASH = 2


def _proj_body(feat_ref, w1_ref, g_ref):
    f = feat_ref[...].astype(jnp.bfloat16)
    w = w1_ref[...].astype(jnp.bfloat16)
    g = jax.lax.dot_general(
        f, w, (((1,), (0,)), ((), ())), preferred_element_type=jnp.float32
    )
    g_ref[...] = g.astype(jnp.bfloat16)


_CHUNKS = (0, 2560, 5120, 7680, 10000)


def _dot_bf16(a, b):
    return jax.lax.dot_general(
        a, b, (((1,), (0,)), ((), ())), preferred_element_type=jnp.float32
    )


def _fused_body(adj_ref, g_ref, b1_ref, w2_ref, out_ref, h2_ref, stash_ref):
    p = pl.program_id(0)
    i = pl.program_id(1)
    first_stashed = _NB - _K_STASH

    @pl.when(p == 0)
    def _phase0():
        acc = _dot_bf16(adj_ref[...], g_ref[...])
        h = jnp.tanh(acc + b1_ref[...])
        h2 = _dot_bf16(h.astype(jnp.bfloat16), w2_ref[...].astype(jnp.bfloat16))
        h2_ref[pl.ds(i * _BM, _BM), :] = h2.astype(jnp.bfloat16)

        @pl.when(i >= first_stashed)
        def _save():
            for c in range(len(_CHUNKS) - 1):
                lo, hi = _CHUNKS[c], _CHUNKS[c + 1]
                stash_ref[pl.ds((i - first_stashed) * _BM, _BM), lo:hi] = (
                    adj_ref[:, lo:hi].astype(jnp.bfloat16)
                )

    @pl.when(jnp.logical_and(p == 1, i < first_stashed))
    def _phase1_stream():
        out_ref[...] = _dot_bf16(adj_ref[...], h2_ref[...])

    @pl.when(jnp.logical_and(p == 1, i >= first_stashed))
    def _phase1_stash():
        a = stash_ref[pl.ds((i - first_stashed) * _BM, _BM), :]
        out_ref[...] = _dot_bf16(a, h2_ref[...])


def _adj_index(p, i):
    first_stashed = _NB - _K_STASH
    streamed = jnp.where(
        jnp.logical_and(p == 1, i >= first_stashed), first_stashed - 1, i
    )
    return (streamed, 0)


@jax.jit
def _run(adj, feat, W1, b1, W2):
    n, d, bm = _N, _D, _BM

    g = pl.pallas_call(
        _proj_body,
        grid=(10,),
        in_specs=[
            pl.BlockSpec((n // 10, d), lambda i: (i, 0)),
            pl.BlockSpec((d, d), lambda i: (0, 0)),
        ],
        out_specs=pl.BlockSpec((n // 10, d), lambda i: (i, 0)),
        out_shape=jax.ShapeDtypeStruct((n, d), jnp.bfloat16),
    )(feat, W1)

    b1_2d = b1.reshape(1, d)

    out = pl.pallas_call(
        _fused_body,
        grid=(2, _NB),
        in_specs=[
            pl.BlockSpec((bm, n), _adj_index),
            pl.BlockSpec((n, d), lambda p, i: (0, 0)),
            pl.BlockSpec((1, d), lambda p, i: (0, 0)),
            pl.BlockSpec((d, d), lambda p, i: (0, 0)),
        ],
        out_specs=pl.BlockSpec((bm, d), lambda p, i: (i, 0)),
        out_shape=jax.ShapeDtypeStruct((n, d), jnp.float32),
        scratch_shapes=[
            pltpu.VMEM((n, d), jnp.bfloat16),
            pltpu.VMEM((_K_STASH * bm, n), jnp.bfloat16),
        ],
        compiler_params=pltpu.CompilerParams(
            vmem_limit_bytes=63 * 1024 * 1024,
        ),
    )(adj, g, b1_2d, W2)
    return out


def kernel(adj, feat, W1, b1, W2):
    return _run(adj, feat, W1, b1, W2)


# speedup vs baseline: 1.1331x; 1.0124x over previous
import functools

import jax
import jax.numpy as jnp
from jax.experimental import pallas as pl
from jax.experimental.pallas import tpu as pltpu

_N = 10000
_D = 128
_BM = 400
_NB = _N // _BM
_K_STASH = 3


def _proj_body(feat_ref, w1_ref, g_ref):
    f = feat_ref[...].astype(jnp.bfloat16)
    w = w1_ref[...].astype(jnp.bfloat16)
    g = jax.lax.dot_general(
        f, w, (((1,), (0,)), ((), ())), preferred_element_type=jnp.float32
    )
    g_ref[...] = g.astype(jnp.bfloat16)


_CHUNKS = (0, 2560, 5120, 7680, 10000)


def _dot_bf16(a, b):
    return jax.lax.dot_general(
        a, b, (((1,), (0,)), ((), ())), preferred_element_type=jnp.float32
    )


def _fused_body(adj_ref, g_ref, b1_ref, w2_ref, out_ref, h2_ref, stash_ref):
    p = pl.program_id(0)
    i = pl.program_id(1)
    first_stashed = _NB - _K_STASH

    @pl.when(p == 0)
    def _phase0():
        acc = _dot_bf16(adj_ref[...], g_ref[...])
        h = jnp.tanh(acc + b1_ref[...])
        h2 = _dot_bf16(h.astype(jnp.bfloat16), w2_ref[...].astype(jnp.bfloat16))
        h2_ref[pl.ds(i * _BM, _BM), :] = h2.astype(jnp.bfloat16)

        @pl.when(i >= first_stashed)
        def _save():
            for c in range(len(_CHUNKS) - 1):
                lo, hi = _CHUNKS[c], _CHUNKS[c + 1]
                stash_ref[pl.ds((i - first_stashed) * _BM, _BM), lo:hi] = (
                    adj_ref[:, lo:hi].astype(jnp.bfloat16)
                )

    @pl.when(jnp.logical_and(p == 1, i < first_stashed))
    def _phase1_stream():
        out_ref[...] = _dot_bf16(adj_ref[...], h2_ref[...])

    @pl.when(jnp.logical_and(p == 1, i >= first_stashed))
    def _phase1_stash():
        a = stash_ref[pl.ds((i - first_stashed) * _BM, _BM), :]
        out_ref[...] = _dot_bf16(a, h2_ref[...])


def _adj_index(p, i):
    first_stashed = _NB - _K_STASH
    streamed = jnp.where(
        jnp.logical_and(p == 1, i >= first_stashed), first_stashed - 1, i
    )
    return (streamed, 0)


@jax.jit
def _run(adj, feat, W1, b1, W2):
    n, d, bm = _N, _D, _BM

    g = pl.pallas_call(
        _proj_body,
        grid=(10,),
        in_specs=[
            pl.BlockSpec((n // 10, d), lambda i: (i, 0)),
            pl.BlockSpec((d, d), lambda i: (0, 0)),
        ],
        out_specs=pl.BlockSpec((n // 10, d), lambda i: (i, 0)),
        out_shape=jax.ShapeDtypeStruct((n, d), jnp.bfloat16),
    )(feat, W1)

    b1_2d = b1.reshape(1, d)

    out = pl.pallas_call(
        _fused_body,
        grid=(2, _NB),
        in_specs=[
            pl.BlockSpec((bm, n), _adj_index),
            pl.BlockSpec((n, d), lambda p, i: (0, 0)),
            pl.BlockSpec((1, d), lambda p, i: (0, 0)),
            pl.BlockSpec((d, d), lambda p, i: (0, 0)),
        ],
        out_specs=pl.BlockSpec((bm, d), lambda p, i: (i, 0)),
        out_shape=jax.ShapeDtypeStruct((n, d), jnp.float32),
        scratch_shapes=[
            pltpu.VMEM((n, d), jnp.bfloat16),
            pltpu.VMEM((_K_STASH * bm, n), jnp.bfloat16),
        ],
        compiler_params=pltpu.CompilerParams(
            vmem_limit_bytes=63 * 1024 * 1024,
        ),
    )(adj, g, b1_2d, W2)
    return out


def kernel(adj, feat, W1, b1, W2):
    return _run(adj, feat, W1, b1, W2)


# speedup vs baseline: 1.1514x; 1.0162x over previous
import functools

import jax
import jax.numpy as jnp
from jax.experimental import pallas as pl
from jax.experimental.pallas import tpu as pltpu

_N = 10000
_D = 128
_BM = 400
_NB = _N // _BM
_K_STASH = 3


def _proj_body(feat_ref, w1_ref, g_ref):
    f = feat_ref[...].astype(jnp.bfloat16)
    w = w1_ref[...].astype(jnp.bfloat16)
    g = jax.lax.dot_general(
        f, w, (((1,), (0,)), ((), ())), preferred_element_type=jnp.float32
    )
    g_ref[...] = g.astype(jnp.bfloat16)


_CHUNKS = (0, 2560, 5120, 7680, 10000)


def _dot_bf16(a, b):
    return jax.lax.dot_general(
        a, b, (((1,), (0,)), ((), ())), preferred_element_type=jnp.float32
    )


def _fused_body(adj_ref, g_ref, b1_ref, w2_ref, out_ref, h2_ref, stash_ref):
    p = pl.program_id(0)
    i = pl.program_id(1)
    first_stashed = _NB - _K_STASH

    @pl.when(p == 0)
    def _phase0():
        acc = _dot_bf16(adj_ref[...], g_ref[...])
        h = jnp.tanh(acc + b1_ref[...])
        h2 = _dot_bf16(h.astype(jnp.bfloat16), w2_ref[...].astype(jnp.bfloat16))
        h2_ref[pl.ds(i * _BM, _BM), :] = h2.astype(jnp.bfloat16)

        @pl.when(i >= first_stashed)
        def _save():
            for c in range(len(_CHUNKS) - 1):
                lo, hi = _CHUNKS[c], _CHUNKS[c + 1]
                stash_ref[pl.ds((i - first_stashed) * _BM, _BM), lo:hi] = (
                    adj_ref[:, lo:hi].astype(jnp.bfloat16)
                )

    @pl.when(jnp.logical_and(p == 1, i < first_stashed))
    def _phase1_stream():
        out_ref[...] = _dot_bf16(adj_ref[...], h2_ref[...])

    @pl.when(jnp.logical_and(p == 1, i >= first_stashed))
    def _phase1_stash():
        a = stash_ref[pl.ds((i - first_stashed) * _BM, _BM), :]
        out_ref[...] = _dot_bf16(a, h2_ref[...])


def _adj_index(p, i):
    first_stashed = _NB - _K_STASH
    streamed = jnp.where(
        jnp.logical_and(p == 1, i >= first_stashed), first_stashed - 1, i
    )
    return (streamed, 0)


@jax.jit
def _run(adj, feat, W1, b1, W2):
    n, d, bm = _N, _D, _BM

    g = pl.pallas_call(
        _proj_body,
        out_shape=jax.ShapeDtypeStruct((n, d), jnp.bfloat16),
    )(feat, W1)

    b1_2d = b1.reshape(1, d)

    out = pl.pallas_call(
        _fused_body,
        grid=(2, _NB),
        in_specs=[
            pl.BlockSpec((bm, n), _adj_index),
            pl.BlockSpec((n, d), lambda p, i: (0, 0)),
            pl.BlockSpec((1, d), lambda p, i: (0, 0)),
            pl.BlockSpec((d, d), lambda p, i: (0, 0)),
        ],
        out_specs=pl.BlockSpec((bm, d), lambda p, i: (i, 0)),
        out_shape=jax.ShapeDtypeStruct((n, d), jnp.float32),
        scratch_shapes=[
            pltpu.VMEM((n, d), jnp.bfloat16),
            pltpu.VMEM((_K_STASH * bm, n), jnp.bfloat16),
        ],
        compiler_params=pltpu.CompilerParams(
            vmem_limit_bytes=63 * 1024 * 1024,
        ),
    )(adj, g, b1_2d, W2)
    return out


def kernel(adj, feat, W1, b1, W2):
    return _run(adj, feat, W1, b1, W2)
